# Initial kernel scaffold; baseline (speedup 1.0000x reference)
import jax
import jax.numpy as jnp
from jax.experimental import pallas as pl


def kernel(pokemon_ids, move_ids, item_ids, tera_ids, P, M, I, T):
    raise NotImplementedError("write your pallas kernel here")


# speedup vs baseline: 3.4984x; 3.4984x over previous
import functools

import jax
import jax.numpy as jnp
from jax import lax
from jax.experimental import pallas as pl
from jax.experimental.pallas import tpu as pltpu
from jax.experimental.pallas import tpu_sc as plsc

B = 16384
NSLOT = 6
SLOTS = B * NSLOT
NC, NS = 2, 16
NW = NC * NS
G = 128
H = 64
SPW = SLOTS // NW
NG = SPW // G
ROW = 120
MROWS = 12544
IROW0 = MROWS
SROWS = 12672

_mesh = plsc.VectorSubcoreMesh(
    core_axis_name="c", subcore_axis_name="s", num_cores=NC, num_subcores=NS
)


@functools.partial(
    pl.kernel,
    out_type=jax.ShapeDtypeStruct((SLOTS * ROW,), jnp.float32),
    mesh=_mesh,
    scratch_types=[
        pltpu.VMEM_SHARED((SROWS, 128), jnp.float32),
        pltpu.VMEM((2, 128), jnp.float32),
        pltpu.VMEM((G,), jnp.int32),
        pltpu.VMEM((4 * G,), jnp.int32),
        pltpu.VMEM((G,), jnp.int32),
        pltpu.VMEM((G,), jnp.int32),
        pltpu.VMEM((2, H), jnp.int32),
        pltpu.VMEM((2, H), jnp.int32),
        pltpu.VMEM((4, 2, H), jnp.int32),
        pltpu.VMEM((4, 2, H), jnp.int32),
        pltpu.VMEM((2, H), jnp.int32),
        pltpu.VMEM((2, H), jnp.int32),
        pltpu.VMEM((H, 128), jnp.float32),
        pltpu.VMEM((H, 128), jnp.float32),
        pltpu.VMEM((H * ROW,), jnp.float32),
        pltpu.SemaphoreType.DMA,
        pltpu.SemaphoreType.DMA,
        pltpu.SemaphoreType.DMA,
    ],
    compiler_params=pltpu.CompilerParams(needs_layout_passes=False),
)
def _encode(pid_h, mid_h, iid_h, tid_h, p4_h, sp_h, t_h, out_h,
            msp, ttab, pg, mg, ig, tg,
            pq, pr, mq, mr, iq, ir, pbuf, mbuf, obuf, psem, msem, wsem):
    sid = lax.axis_index("s")
    cid = lax.axis_index("c")
    wid = sid * NC + cid

    srows = SROWS // NS
    pltpu.sync_copy(sp_h.at[pl.ds(sid * srows, srows)],
                    msp.at[pl.ds(sid * srows, srows)])
    pltpu.sync_copy(t_h, ttab)
    plsc.subcore_barrier()

    iota = lax.iota(jnp.int32, 16)

    @pl.loop(0, NG)
    def _(g):
        base = wid * SPW + g * G

        pltpu.sync_copy(pid_h.at[pl.ds(base, G)], pg)
        pltpu.sync_copy(mid_h.at[pl.ds(4 * base, 4 * G)], mg)
        pltpu.sync_copy(iid_h.at[pl.ds(base, G)], ig)
        pltpu.sync_copy(tid_h.at[pl.ds(base, G)], tg)

        for c in range(8):
            h, cl = c // 4, c % 4
            sl = pl.ds(16 * cl, 16)
            pv = pg[pl.ds(16 * c, 16)]
            pq[h, sl] = lax.shift_right_logical(pv, 2)
            pr[h, sl] = lax.bitwise_and(pv, 3)
            iv = ig[pl.ds(16 * c, 16)]
            iq[h, sl] = IROW0 + lax.shift_right_logical(iv, 3)
            ir[h, sl] = lax.bitwise_and(iv, 7)
            for q in range(4):
                mv = plsc.load_gather(mg, [(iota + 16 * c) * 4 + q])
                mq[q, h, sl] = lax.shift_right_logical(mv, 3)
                mr[q, h, sl] = lax.bitwise_and(mv, 7)

        for h in range(2):
            pcopy = pltpu.async_copy(p4_h.at[pq.at[h]], pbuf, psem)

            for q in range(4):
                mcopy = pltpu.async_copy(msp.at[mq.at[q, h]], mbuf, msem)
                mcopy.wait()
                for cl in range(4):
                    rows = iota + 16 * cl
                    rowbase = rows * ROW + 32 + 16 * q
                    col0 = mr[q, h, pl.ds(16 * cl, 16)] * 16

                    @pl.loop(0, 16, unroll=2)
                    def _(d):
                        vals = plsc.load_gather(mbuf, [rows, col0 + d])
                        plsc.store_scatter(obuf, [rowbase + d], vals)

            icopy = pltpu.async_copy(msp.at[iq.at[h]], mbuf, msem)
            icopy.wait()
            for cl in range(4):
                rows = iota + 16 * cl
                rowbase = rows * ROW + 96
                col0 = ir[h, pl.ds(16 * cl, 16)] * 16

                @pl.loop(0, 16, unroll=2)
                def _(d):
                    vals = plsc.load_gather(mbuf, [rows, col0 + d])
                    plsc.store_scatter(obuf, [rowbase + d], vals)

            pcopy.wait()
            for cl in range(4):
                rows = iota + 16 * cl
                rowbase = rows * ROW
                col0 = pr[h, pl.ds(16 * cl, 16)] * 32

                @pl.loop(0, 32, unroll=2)
                def _(d):
                    vals = plsc.load_gather(pbuf, [rows, col0 + d])
                    plsc.store_scatter(obuf, [rowbase + d], vals)

            for cl in range(4):
                rows = iota + 16 * cl
                rowbase = rows * ROW + 112
                tflat = tg[pl.ds(64 * h + 16 * cl, 16)] * 8

                @pl.loop(0, 8, unroll=2)
                def _(d):
                    fl = tflat + d
                    vals = plsc.load_gather(
                        ttab, [lax.shift_right_logical(fl, 7),
                               lax.bitwise_and(fl, 127)])
                    plsc.store_scatter(obuf, [rowbase + d], vals)

            ob = (base + H * h) * ROW
            pltpu.async_copy(obuf, out_h.at[pl.ds(ob, H * ROW)], wsem)
            pltpu.make_async_copy(obuf, out_h.at[pl.ds(ob, H * ROW)],
                                  wsem).wait()


def kernel(pokemon_ids, move_ids, item_ids, tera_ids, P, M, I, T):
    pid = pokemon_ids.astype(jnp.int32).reshape(SLOTS)
    mid = move_ids.astype(jnp.int32).reshape(SLOTS * 4)
    iid = item_ids.astype(jnp.int32).reshape(SLOTS)
    tid = tera_ids.astype(jnp.int32).reshape(SLOTS)
    p4 = P.reshape(25000, 128)
    sp = jnp.concatenate([
        jnp.pad(M.reshape(12500, 128), ((0, MROWS - 12500), (0, 0))),
        jnp.pad(I.reshape(125, 128), ((0, SROWS - IROW0 - 125), (0, 0))),
    ])
    t2 = jnp.pad(T.reshape(-1), (0, 96)).reshape(2, 128)
    out = _encode(pid, mid, iid, tid, p4, sp, t2)
    return out.reshape(B, NSLOT * ROW)


# speedup vs baseline: 4.2014x; 1.2009x over previous
import functools

import jax
import jax.numpy as jnp
from jax import lax
from jax.experimental import pallas as pl
from jax.experimental.pallas import tpu as pltpu
from jax.experimental.pallas import tpu_sc as plsc

B = 16384
NSLOT = 6
SLOTS = B * NSLOT
NC, NS = 2, 16
NW = NC * NS
Q = 32
SPW = SLOTS // NW
NQ = SPW // Q
ROW = 120
MROWS = 12544
IROW0 = MROWS
SROWS = 12672

_mesh = plsc.VectorSubcoreMesh(
    core_axis_name="c", subcore_axis_name="s", num_cores=NC, num_subcores=NS
)


@functools.partial(
    pl.kernel,
    out_type=jax.ShapeDtypeStruct((SLOTS * ROW,), jnp.float32),
    mesh=_mesh,
    scratch_types=[
        pltpu.VMEM_SHARED((SROWS, 128), jnp.float32),
        pltpu.VMEM((2, 128), jnp.float32),
        pltpu.VMEM((2 * Q,), jnp.int32),
        pltpu.VMEM((2 * 4 * Q,), jnp.int32),
        pltpu.VMEM((2 * Q,), jnp.int32),
        pltpu.VMEM((2 * Q,), jnp.int32),
        pltpu.VMEM((2, Q), jnp.int32),
        pltpu.VMEM((2, Q), jnp.int32),
        pltpu.VMEM((2, 4, Q), jnp.int32),
        pltpu.VMEM((2, 4, Q), jnp.int32),
        pltpu.VMEM((2, Q), jnp.int32),
        pltpu.VMEM((2, Q), jnp.int32),
        pltpu.VMEM((Q, 128), jnp.float32),
        pltpu.VMEM((2 * Q, 128), jnp.float32),
        pltpu.VMEM((2 * Q * ROW,), jnp.float32),
        pltpu.SemaphoreType.DMA,
        pltpu.SemaphoreType.DMA,
        pltpu.SemaphoreType.DMA,
        pltpu.SemaphoreType.DMA,
        pltpu.SemaphoreType.DMA,
        pltpu.SemaphoreType.DMA,
        pltpu.SemaphoreType.DMA,
    ],
    compiler_params=pltpu.CompilerParams(needs_layout_passes=False),
)
def _encode(pid_h, mid_h, iid_h, tid_h, p4_h, sp_h, t_h, out_h,
            msp, ttab, pg, mg, ig, tg, pq, pr, mq, mr, iq, ir,
            pbuf, mbuf, obuf,
            isem0, isem1, psem, msem0, msem1, wsem0, wsem1):
    sid = lax.axis_index("s")
    cid = lax.axis_index("c")
    wid = sid * NC + cid
    isems = (isem0, isem1)
    msems = (msem0, msem1)
    wsems = (wsem0, wsem1)

    srows = SROWS // NS
    pltpu.sync_copy(sp_h.at[pl.ds(sid * srows, srows)],
                    msp.at[pl.ds(sid * srows, srows)])
    pltpu.sync_copy(t_h, ttab)
    plsc.subcore_barrier()

    iota = lax.iota(jnp.int32, 16)

    def idx_list(k, b):
        base = wid * SPW + k * Q
        return [
            (pid_h.at[pl.ds(base, Q)], pg.at[pl.ds(b * Q, Q)]),
            (mid_h.at[pl.ds(4 * base, 4 * Q)],
             mg.at[pl.ds(b * 4 * Q, 4 * Q)]),
            (iid_h.at[pl.ds(base, Q)], ig.at[pl.ds(b * Q, Q)]),
            (tid_h.at[pl.ds(base, Q)], tg.at[pl.ds(b * Q, Q)]),
        ]

    def stage_idx(k, b):
        for src, dst in idx_list(k, b):
            pltpu.async_copy(src, dst, isems[b])

    def wait_idx(k, b):
        for src, dst in idx_list(k, b):
            pltpu.make_async_copy(src, dst, isems[b]).wait()

    def splits(b):
        for c in range(2):
            sl = pl.ds(16 * c, 16)
            fsl = pl.ds(b * Q + 16 * c, 16)
            pv = pg[fsl]
            pq[b, sl] = lax.shift_right_logical(pv, 2)
            pr[b, sl] = lax.bitwise_and(pv, 3)
            iv = ig[fsl]
            iq[b, sl] = IROW0 + lax.shift_right_logical(iv, 3)
            ir[b, sl] = lax.bitwise_and(iv, 7)
            for q in range(4):
                mv = plsc.load_gather(
                    mg, [b * 4 * Q + (iota + 16 * c) * 4 + q])
                mq[b, q, sl] = lax.shift_right_logical(mv, 3)
                mr[b, q, sl] = lax.bitwise_and(mv, 7)

    def sub_src(s, b):
        return msp.at[mq.at[b, s]] if s < 4 else msp.at[iq.at[b]]

    def extract_sub(s, b, mb):
        off = 32 + 16 * s if s < 4 else 96
        for c in range(2):
            rows = iota + 16 * c
            rowbase = rows * ROW + (b * Q * ROW + off)
            srows = rows + mb * Q
            sl = pl.ds(16 * c, 16)
            col0 = (mr[b, s, sl] if s < 4 else ir[b, sl]) * 16

            @pl.loop(0, 16, unroll=4)
            def _(d):
                vals = plsc.load_gather(mbuf, [srows, col0 + d])
                plsc.store_scatter(obuf, [rowbase + d], vals)

    def out_slice(k):
        return out_h.at[pl.ds((wid * SPW + k * Q) * ROW, Q * ROW)]

    stage_idx(0, 0)

    @pl.loop(0, NQ, step=2)
    def _(k0):
        for bb in range(2):
            k = k0 + bb
            b, b2 = bb, 1 - bb

            wait_idx(k, b)
            splits(b)

            @pl.when(k + 1 < NQ)
            def _():
                stage_idx(k + 1, b2)

            pltpu.async_copy(p4_h.at[pq.at[b]], pbuf, psem)
            pltpu.async_copy(sub_src(0, b), mbuf.at[pl.ds(0, Q)], msems[0])

            @pl.when(k >= 2)
            def _():
                pltpu.make_async_copy(obuf.at[pl.ds(b * Q * ROW, Q * ROW)],
                                      out_slice(k - 2), wsems[b]).wait()

            for s in range(5):
                if s < 4:
                    pltpu.async_copy(sub_src(s + 1, b),
                                     mbuf.at[pl.ds(((s + 1) % 2) * Q, Q)],
                                     msems[(s + 1) % 2])
                pltpu.make_async_copy(sub_src(s, b),
                                      mbuf.at[pl.ds((s % 2) * Q, Q)],
                                      msems[s % 2]).wait()
                extract_sub(s, b, s % 2)

            pltpu.make_async_copy(p4_h.at[pq.at[b]], pbuf, psem).wait()
            for c in range(2):
                rows = iota + 16 * c
                rowbase = rows * ROW + b * Q * ROW
                col0 = pr[b, pl.ds(16 * c, 16)] * 32

                @pl.loop(0, 32, unroll=4)
                def _(d):
                    vals = plsc.load_gather(pbuf, [rows, col0 + d])
                    plsc.store_scatter(obuf, [rowbase + d], vals)

            for c in range(2):
                rows = iota + 16 * c
                rowbase = rows * ROW + (b * Q * ROW + 112)
                tflat = tg[pl.ds(b * Q + 16 * c, 16)] * 8

                @pl.loop(0, 8, unroll=4)
                def _(d):
                    fl = tflat + d
                    vals = plsc.load_gather(
                        ttab, [lax.shift_right_logical(fl, 7),
                               lax.bitwise_and(fl, 127)])
                    plsc.store_scatter(obuf, [rowbase + d], vals)

            pltpu.async_copy(obuf.at[pl.ds(b * Q * ROW, Q * ROW)],
                             out_slice(k), wsems[b])

    pltpu.make_async_copy(obuf.at[pl.ds(0, Q * ROW)],
                          out_slice(NQ - 2), wsems[0]).wait()
    pltpu.make_async_copy(obuf.at[pl.ds(Q * ROW, Q * ROW)],
                          out_slice(NQ - 1), wsems[1]).wait()


def kernel(pokemon_ids, move_ids, item_ids, tera_ids, P, M, I, T):
    pid = pokemon_ids.astype(jnp.int32).reshape(SLOTS)
    mid = move_ids.astype(jnp.int32).reshape(SLOTS * 4)
    iid = item_ids.astype(jnp.int32).reshape(SLOTS)
    tid = tera_ids.astype(jnp.int32).reshape(SLOTS)
    p4 = P.reshape(25000, 128)
    sp = jnp.concatenate([
        jnp.pad(M.reshape(12500, 128), ((0, MROWS - 12500), (0, 0))),
        jnp.pad(I.reshape(125, 128), ((0, SROWS - IROW0 - 125), (0, 0))),
    ])
    t2 = jnp.pad(T.reshape(-1), (0, 96)).reshape(2, 128)
    out = _encode(pid, mid, iid, tid, p4, sp, t2)
    return out.reshape(B, NSLOT * ROW)
